# baseline (speedup 1.0000x reference)
import functools

import jax
import jax.numpy as jnp
from jax import lax
from jax.experimental import pallas as pl
from jax.experimental.pallas import tpu as pltpu
from jax.experimental.pallas import tpu_sc as plsc

N = 10000
E = 320000
D = 128

NC = 2
NS = 16
NW = NC * NS
EPT = E // NW
C = 40
NCHUNK = EPT // C
RPT = 632
N_PAD = RPT * NS

_LN_EPS = 1e-5


def _node_proj_body(ne_ref, ws_ref, wt_ref, ns_ref, nt_ref):
    x = ne_ref[...]
    ns_ref[...] = jnp.dot(x, ws_ref[...], preferred_element_type=jnp.float32)
    nt_ref[...] = jnp.dot(x, wt_ref[...], preferred_element_type=jnp.float32)


def _node_proj(node_emb, w_s, w_t):
    return pl.pallas_call(
        _node_proj_body,
        out_shape=[
            jax.ShapeDtypeStruct((N, D), jnp.float32),
            jax.ShapeDtypeStruct((N, D), jnp.float32),
        ],
    )(node_emb, w_s, w_t)


_BE = 3200


def _edge_proj_body(ee_ref, w_ref, z_ref):
    z_ref[...] = jnp.dot(ee_ref[...], w_ref[...], preferred_element_type=jnp.float32)


def _edge_proj(edge_emb, w_e):
    return pl.pallas_call(
        _edge_proj_body,
        grid=(E // _BE,),
        in_specs=[
            pl.BlockSpec((_BE, D), lambda i: (i, 0)),
            pl.BlockSpec((D, D), lambda i: (0, 0)),
        ],
        out_specs=pl.BlockSpec((_BE, D), lambda i: (i, 0)),
        out_shape=jax.ShapeDtypeStruct((E, D), jnp.float32),
    )(edge_emb, w_e)


def _lane_gather(v, perm):
    dnums = lax.GatherDimensionNumbers(
        offset_dims=(), collapsed_slice_dims=(0,), start_index_map=(0,))
    return lax.gather(v, perm[:, None], dnums, (1,),
                      mode=lax.GatherScatterMode.PROMISE_IN_BOUNDS)


_UNROLL = 2


def _sc_body(src_hbm, dst_hbm, ns_hbm, nt_hbm, z_hbm, g_hbm, b_hbm, zeros_hbm,
             out_hbm,
             idx_s0, idx_d0, idx_s1, idx_d1, idx_s2, idx_d2, idx_s3, idx_d3,
             rows_s0, rows_t0, rows_z0, out_buf0,
             rows_s1, rows_t1, rows_z1, out_buf1,
             gb_buf, agg,
             sem_i0, sem_i1, sem_i2, sem_i3,
             sem_s0, sem_t0, sem_z0, sem_s1, sem_t1, sem_z1):
    cid = lax.axis_index("c")
    sid = lax.axis_index("s")
    wid = cid * NS + sid
    ebase = wid * EPT

    idxb = ((idx_s0, idx_d0, sem_i0), (idx_s1, idx_d1, sem_i1),
            (idx_s2, idx_d2, sem_i2), (idx_s3, idx_d3, sem_i3))
    rowb = ((rows_s0, rows_t0, rows_z0, out_buf0, sem_s0, sem_t0, sem_z0),
            (rows_s1, rows_t1, rows_z1, out_buf1, sem_s1, sem_t1, sem_z1))

    pltpu.sync_copy(zeros_hbm.at[pl.ds(sid * RPT, RPT)],
                    agg.at[pl.ds(sid * RPT, RPT)])
    pltpu.sync_copy(g_hbm, gb_buf.at[0])
    pltpu.sync_copy(b_hbm, gb_buf.at[1])
    gvec = tuple(gb_buf[0, pl.ds(k * 16, 16)] for k in range(8))
    bvec = tuple(gb_buf[1, pl.ds(k * 16, 16)] for k in range(8))

    def issue_idx(j, q):
        idx_s, idx_d, sem_i = idxb[q]
        base = ebase + j * C
        pltpu.async_copy(src_hbm.at[pl.ds(base, C)], idx_s, sem_i)
        pltpu.async_copy(dst_hbm.at[pl.ds(base, C)], idx_d, sem_i)

    def wait_idx(j, q):
        idx_s, idx_d, sem_i = idxb[q]
        base = ebase + j * C
        pltpu.make_async_copy(src_hbm.at[pl.ds(base, C)], idx_s, sem_i).wait()
        pltpu.make_async_copy(dst_hbm.at[pl.ds(base, C)], idx_d, sem_i).wait()

    def issue_gathers(j, q, b):
        idx_s, idx_d, _ = idxb[q]
        rows_s, rows_t, rows_z, _, sem_s, sem_t, sem_z = rowb[b]
        pltpu.async_copy(ns_hbm.at[idx_s], rows_s, sem_s)
        pltpu.async_copy(nt_hbm.at[idx_d], rows_t, sem_t)
        pltpu.async_copy(z_hbm.at[pl.ds(ebase + j * C, C)], rows_z, sem_z)

    def wait_gathers(j, q, b):
        idx_s, idx_d, _ = idxb[q]
        rows_s, rows_t, rows_z, _, sem_s, sem_t, sem_z = rowb[b]
        pltpu.make_async_copy(ns_hbm.at[idx_s], rows_s, sem_s).wait()
        pltpu.make_async_copy(nt_hbm.at[idx_d], rows_t, sem_t).wait()
        pltpu.make_async_copy(z_hbm.at[pl.ds(ebase + j * C, C)], rows_z,
                              sem_z).wait()

    def edge_chunk(q, b):
        _, idx_d, _ = idxb[q]
        rows_s, rows_t, rows_z, out_buf, _, _, _ = rowb[b]

        @plsc.parallel_loop(0, C, step=1, unroll=_UNROLL)
        def edge_group(e):
            xs = []
            for k in range(8):
                sl = pl.ds(k * 16, 16)
                x = rows_s[e, sl] + rows_t[e, sl] + rows_z[e, sl]
                xs.append(x / (1.0 + jnp.exp(-x)))
            tot = xs[0]
            sq = xs[0] * xs[0]
            for k in range(1, 8):
                tot = tot + xs[k]
                sq = sq + xs[k] * xs[k]
            for sh in (8, 4, 2, 1):
                perm = lax.iota(jnp.int32, 16) ^ sh
                tot = tot + _lane_gather(tot, perm)
                sq = sq + _lane_gather(sq, perm)
            mean = tot * (1.0 / D)
            ex2 = sq * (1.0 / D)
            var = ex2 - mean * mean + _LN_EPS
            bits = lax.bitcast_convert_type(var, jnp.int32)
            r = lax.bitcast_convert_type(
                jnp.int32(0x5F3759DF) - lax.shift_right_arithmetic(bits, 1),
                jnp.float32)
            for _ in range(2):
                r = r * (1.5 - 0.5 * var * r * r)
            for k in range(8):
                sl = pl.ds(k * 16, 16)
                out_buf[e, sl] = ((xs[k] - mean) * r) * gvec[k] + bvec[k]

        pltpu.sync_copy(out_buf, agg.at[idx_d], add=True)

    for q in range(4):
        issue_idx(q, q)
    wait_idx(0, 0)
    issue_gathers(0, 0, 0)
    plsc.subcore_barrier()

    LAST = NCHUNK - 1

    def quad_body(qq, carry):
        j0 = qq * 4
        for pos in range(4):
            j = j0 + pos
            b = pos % 2
            wait_idx(j + 1, (pos + 1) % 4)
            issue_gathers(j + 1, (pos + 1) % 4, 1 - b)
            wait_gathers(j, pos, b)
            edge_chunk(pos, b)
            issue_idx(jnp.minimum(j + 4, LAST), pos)
        return carry

    lax.fori_loop(0, NCHUNK // 4, quad_body, 0)

    wait_idx(LAST, 1)
    issue_gathers(LAST, 1, 1)
    wait_gathers(LAST - 1, 0, 0)
    edge_chunk(0, 0)
    wait_gathers(LAST, 1, 1)
    edge_chunk(1, 1)
    wait_idx(LAST, 2)
    wait_idx(LAST, 3)

    plsc.subcore_barrier()
    pltpu.sync_copy(agg.at[pl.ds(sid * RPT, RPT)],
                    out_hbm.at[cid, pl.ds(sid * RPT, RPT)])


_sc_call = pl.kernel(
    _sc_body,
    out_type=jax.ShapeDtypeStruct((NC, N_PAD, D), jnp.float32),
    mesh=plsc.VectorSubcoreMesh(core_axis_name="c", subcore_axis_name="s"),
    scratch_types=(
        [pltpu.VMEM((C,), jnp.int32)] * 8
        + [pltpu.VMEM((C, D), jnp.float32)] * 8
        + [pltpu.VMEM((2, D), jnp.float32),
           pltpu.VMEM_SHARED((N_PAD, D), jnp.float32)]
        + [pltpu.SemaphoreType.DMA] * 10
    ),
)


def _final_body(p_ref, ne_ref, we_ref, wt_ref, g_ref, b_ref, out_ref):
    aggv = p_ref[0] + p_ref[1]
    t = (jnp.dot(aggv, we_ref[...], preferred_element_type=jnp.float32)
         + jnp.dot(ne_ref[...], wt_ref[...], preferred_element_type=jnp.float32))
    t = t / (1.0 + jnp.exp(-t))
    mu = jnp.mean(t, axis=1, keepdims=True)
    d = t - mu
    var = jnp.mean(d * d, axis=1, keepdims=True)
    out_ref[...] = d * lax.rsqrt(var + _LN_EPS) * g_ref[...] + b_ref[...]


def _final(parts, node_emb, w_e2t, w_t2t, g2, b2):
    return pl.pallas_call(
        _final_body,
        out_shape=jax.ShapeDtypeStruct((N, D), jnp.float32),
    )(parts, node_emb, w_e2t, w_t2t, g2, b2)


def kernel(node_emb, edge_emb, edge_index, W_s2e, W_t2e, W_e2e, W_e2t, W_t2t,
           g1, b1, g2, b2):
    src = edge_index[0]
    dst = edge_index[1]
    node_s, node_t = _node_proj(node_emb, W_s2e, W_t2e)
    z = _edge_proj(edge_emb, W_e2e)
    zeros = jnp.zeros((N_PAD, D), jnp.float32)
    parts = _sc_call(src, dst, node_s, node_t, z, g1, b1, zeros)
    parts = parts[:, :N, :]
    return _final(parts, node_emb, W_e2t, W_t2t,
                  g2.reshape(1, D), b2.reshape(1, D))


# speedup vs baseline: 1.0036x; 1.0036x over previous
import functools

import jax
import jax.numpy as jnp
from jax import lax
from jax.experimental import pallas as pl
from jax.experimental.pallas import tpu as pltpu
from jax.experimental.pallas import tpu_sc as plsc

N = 10000
E = 320000
D = 128

NC = 2
NS = 16
NW = NC * NS
EPT = E // NW
C = 40
NCHUNK = EPT // C
RPT = 632
N_PAD = RPT * NS

_LN_EPS = 1e-5


_BE = 3200


def _proj_body(ee_ref, we_ref, ne_ref, ws_ref, wt_ref, z_ref, ns_ref, nt_ref):
    z_ref[...] = jnp.dot(ee_ref[...], we_ref[...],
                         preferred_element_type=jnp.float32)

    @pl.when(pl.program_id(0) == 0)
    def _():
        x = ne_ref[...]
        ns_ref[...] = jnp.dot(x, ws_ref[...], preferred_element_type=jnp.float32)
        nt_ref[...] = jnp.dot(x, wt_ref[...], preferred_element_type=jnp.float32)


def _projections(edge_emb, w_e, node_emb, w_s, w_t):
    full = lambda i: (0, 0)
    return pl.pallas_call(
        _proj_body,
        grid=(E // _BE,),
        in_specs=[
            pl.BlockSpec((_BE, D), lambda i: (i, 0)),
            pl.BlockSpec((D, D), full),
            pl.BlockSpec((N, D), full),
            pl.BlockSpec((D, D), full),
            pl.BlockSpec((D, D), full),
        ],
        out_specs=[
            pl.BlockSpec((_BE, D), lambda i: (i, 0)),
            pl.BlockSpec((N, D), full),
            pl.BlockSpec((N, D), full),
        ],
        out_shape=[
            jax.ShapeDtypeStruct((E, D), jnp.float32),
            jax.ShapeDtypeStruct((N, D), jnp.float32),
            jax.ShapeDtypeStruct((N, D), jnp.float32),
        ],
    )(edge_emb, w_e, node_emb, w_s, w_t)


def _lane_gather(v, perm):
    dnums = lax.GatherDimensionNumbers(
        offset_dims=(), collapsed_slice_dims=(0,), start_index_map=(0,))
    return lax.gather(v, perm[:, None], dnums, (1,),
                      mode=lax.GatherScatterMode.PROMISE_IN_BOUNDS)


_UNROLL = 2


def _sc_body(src_hbm, dst_hbm, ns_hbm, nt_hbm, z_hbm, g_hbm, b_hbm, zeros_hbm,
             out_hbm,
             idx_s0, idx_d0, idx_s1, idx_d1, idx_s2, idx_d2, idx_s3, idx_d3,
             rows_s0, rows_t0, rows_z0, out_buf0,
             rows_s1, rows_t1, rows_z1, out_buf1,
             gb_buf, agg,
             sem_i0, sem_i1, sem_i2, sem_i3,
             sem_s0, sem_t0, sem_z0, sem_s1, sem_t1, sem_z1):
    cid = lax.axis_index("c")
    sid = lax.axis_index("s")
    wid = cid * NS + sid
    ebase = wid * EPT

    idxb = ((idx_s0, idx_d0, sem_i0), (idx_s1, idx_d1, sem_i1),
            (idx_s2, idx_d2, sem_i2), (idx_s3, idx_d3, sem_i3))
    rowb = ((rows_s0, rows_t0, rows_z0, out_buf0, sem_s0, sem_t0, sem_z0),
            (rows_s1, rows_t1, rows_z1, out_buf1, sem_s1, sem_t1, sem_z1))

    pltpu.sync_copy(zeros_hbm.at[pl.ds(sid * RPT, RPT)],
                    agg.at[pl.ds(sid * RPT, RPT)])
    pltpu.sync_copy(g_hbm, gb_buf.at[0])
    pltpu.sync_copy(b_hbm, gb_buf.at[1])
    gvec = tuple(gb_buf[0, pl.ds(k * 16, 16)] for k in range(8))
    bvec = tuple(gb_buf[1, pl.ds(k * 16, 16)] for k in range(8))

    def issue_idx(j, q):
        idx_s, idx_d, sem_i = idxb[q]
        base = ebase + j * C
        pltpu.async_copy(src_hbm.at[pl.ds(base, C)], idx_s, sem_i)
        pltpu.async_copy(dst_hbm.at[pl.ds(base, C)], idx_d, sem_i)

    def wait_idx(j, q):
        idx_s, idx_d, sem_i = idxb[q]
        base = ebase + j * C
        pltpu.make_async_copy(src_hbm.at[pl.ds(base, C)], idx_s, sem_i).wait()
        pltpu.make_async_copy(dst_hbm.at[pl.ds(base, C)], idx_d, sem_i).wait()

    def issue_gathers(j, q, b):
        idx_s, idx_d, _ = idxb[q]
        rows_s, rows_t, rows_z, _, sem_s, sem_t, sem_z = rowb[b]
        pltpu.async_copy(ns_hbm.at[idx_s], rows_s, sem_s)
        pltpu.async_copy(nt_hbm.at[idx_d], rows_t, sem_t)
        pltpu.async_copy(z_hbm.at[pl.ds(ebase + j * C, C)], rows_z, sem_z)

    def wait_gathers(j, q, b):
        idx_s, idx_d, _ = idxb[q]
        rows_s, rows_t, rows_z, _, sem_s, sem_t, sem_z = rowb[b]
        pltpu.make_async_copy(ns_hbm.at[idx_s], rows_s, sem_s).wait()
        pltpu.make_async_copy(nt_hbm.at[idx_d], rows_t, sem_t).wait()
        pltpu.make_async_copy(z_hbm.at[pl.ds(ebase + j * C, C)], rows_z,
                              sem_z).wait()

    def edge_chunk(q, b):
        _, idx_d, _ = idxb[q]
        rows_s, rows_t, rows_z, out_buf, _, _, _ = rowb[b]

        @plsc.parallel_loop(0, C, step=1, unroll=_UNROLL)
        def edge_group(e):
            xs = []
            for k in range(8):
                sl = pl.ds(k * 16, 16)
                x = rows_s[e, sl] + rows_t[e, sl] + rows_z[e, sl]
                xs.append(x / (1.0 + jnp.exp(-x)))
            tot = xs[0]
            sq = xs[0] * xs[0]
            for k in range(1, 8):
                tot = tot + xs[k]
                sq = sq + xs[k] * xs[k]
            for sh in (8, 4, 2, 1):
                perm = lax.iota(jnp.int32, 16) ^ sh
                tot = tot + _lane_gather(tot, perm)
                sq = sq + _lane_gather(sq, perm)
            mean = tot * (1.0 / D)
            ex2 = sq * (1.0 / D)
            var = ex2 - mean * mean + _LN_EPS
            bits = lax.bitcast_convert_type(var, jnp.int32)
            r = lax.bitcast_convert_type(
                jnp.int32(0x5F3759DF) - lax.shift_right_arithmetic(bits, 1),
                jnp.float32)
            for _ in range(2):
                r = r * (1.5 - 0.5 * var * r * r)
            for k in range(8):
                sl = pl.ds(k * 16, 16)
                out_buf[e, sl] = ((xs[k] - mean) * r) * gvec[k] + bvec[k]

        pltpu.sync_copy(out_buf, agg.at[idx_d], add=True)

    for q in range(4):
        issue_idx(q, q)
    wait_idx(0, 0)
    issue_gathers(0, 0, 0)
    plsc.subcore_barrier()

    LAST = NCHUNK - 1

    def quad_body(qq, carry):
        j0 = qq * 4
        for pos in range(4):
            j = j0 + pos
            b = pos % 2
            wait_idx(j + 1, (pos + 1) % 4)
            issue_gathers(j + 1, (pos + 1) % 4, 1 - b)
            wait_gathers(j, pos, b)
            edge_chunk(pos, b)
            issue_idx(jnp.minimum(j + 4, LAST), pos)
        return carry

    lax.fori_loop(0, NCHUNK // 4, quad_body, 0)

    wait_idx(LAST, 1)
    issue_gathers(LAST, 1, 1)
    wait_gathers(LAST - 1, 0, 0)
    edge_chunk(0, 0)
    wait_gathers(LAST, 1, 1)
    edge_chunk(1, 1)
    wait_idx(LAST, 2)
    wait_idx(LAST, 3)

    plsc.subcore_barrier()
    pltpu.sync_copy(agg.at[pl.ds(sid * RPT, RPT)],
                    out_hbm.at[cid, pl.ds(sid * RPT, RPT)])


_sc_call = pl.kernel(
    _sc_body,
    out_type=jax.ShapeDtypeStruct((NC, N_PAD, D), jnp.float32),
    mesh=plsc.VectorSubcoreMesh(core_axis_name="c", subcore_axis_name="s"),
    scratch_types=(
        [pltpu.VMEM((C,), jnp.int32)] * 8
        + [pltpu.VMEM((C, D), jnp.float32)] * 8
        + [pltpu.VMEM((2, D), jnp.float32),
           pltpu.VMEM_SHARED((N_PAD, D), jnp.float32)]
        + [pltpu.SemaphoreType.DMA] * 10
    ),
)


def _final_body(p_ref, ne_ref, we_ref, wt_ref, g_ref, b_ref, out_ref):
    aggv = p_ref[0] + p_ref[1]
    t = (jnp.dot(aggv, we_ref[...], preferred_element_type=jnp.float32)
         + jnp.dot(ne_ref[...], wt_ref[...], preferred_element_type=jnp.float32))
    t = t / (1.0 + jnp.exp(-t))
    mu = jnp.mean(t, axis=1, keepdims=True)
    d = t - mu
    var = jnp.mean(d * d, axis=1, keepdims=True)
    out_ref[...] = d * lax.rsqrt(var + _LN_EPS) * g_ref[...] + b_ref[...]


def _final(parts, node_emb, w_e2t, w_t2t, g2, b2):
    return pl.pallas_call(
        _final_body,
        out_shape=jax.ShapeDtypeStruct((N, D), jnp.float32),
    )(parts, node_emb, w_e2t, w_t2t, g2, b2)


def kernel(node_emb, edge_emb, edge_index, W_s2e, W_t2e, W_e2e, W_e2t, W_t2t,
           g1, b1, g2, b2):
    src = edge_index[0]
    dst = edge_index[1]
    z, node_s, node_t = _projections(edge_emb, W_e2e, node_emb, W_s2e, W_t2e)
    zeros = jnp.zeros((N_PAD, D), jnp.float32)
    parts = _sc_call(src, dst, node_s, node_t, z, g1, b1, zeros)
    parts = parts[:, :N, :]
    return _final(parts, node_emb, W_e2t, W_t2t,
                  g2.reshape(1, D), b2.reshape(1, D))


# speedup vs baseline: 1.0136x; 1.0099x over previous
import functools

import jax
import jax.numpy as jnp
from jax import lax
from jax.experimental import pallas as pl
from jax.experimental.pallas import tpu as pltpu
from jax.experimental.pallas import tpu_sc as plsc

N = 10000
E = 320000
D = 128

NC = 2
NS = 16
NW = NC * NS
EPT = E // NW
C = 40
NCHUNK = EPT // C
RPT = 632
N_PAD = RPT * NS

_LN_EPS = 1e-5


_BE = 3200


def _proj_body(ee_ref, we_ref, ne_ref, ws_ref, wt_ref, z_ref, ns_ref, nt_ref):
    z_ref[...] = jnp.dot(ee_ref[...], we_ref[...],
                         preferred_element_type=jnp.float32)

    @pl.when(pl.program_id(0) == 0)
    def _():
        x = ne_ref[...]
        ns_ref[...] = jnp.dot(x, ws_ref[...], preferred_element_type=jnp.float32)
        nt_ref[...] = jnp.dot(x, wt_ref[...], preferred_element_type=jnp.float32)


def _projections(edge_emb, w_e, node_emb, w_s, w_t):
    full = lambda i: (0, 0)
    return pl.pallas_call(
        _proj_body,
        grid=(E // _BE,),
        in_specs=[
            pl.BlockSpec((_BE, D), lambda i: (i, 0)),
            pl.BlockSpec((D, D), full),
            pl.BlockSpec((N, D), full),
            pl.BlockSpec((D, D), full),
            pl.BlockSpec((D, D), full),
        ],
        out_specs=[
            pl.BlockSpec((_BE, D), lambda i: (i, 0)),
            pl.BlockSpec((N, D), full),
            pl.BlockSpec((N, D), full),
        ],
        out_shape=[
            jax.ShapeDtypeStruct((E, D), jnp.float32),
            jax.ShapeDtypeStruct((N, D), jnp.float32),
            jax.ShapeDtypeStruct((N, D), jnp.float32),
        ],
    )(edge_emb, w_e, node_emb, w_s, w_t)


def _lane_gather(v, perm):
    dnums = lax.GatherDimensionNumbers(
        offset_dims=(), collapsed_slice_dims=(0,), start_index_map=(0,))
    return lax.gather(v, perm[:, None], dnums, (1,),
                      mode=lax.GatherScatterMode.PROMISE_IN_BOUNDS)


_UNROLL = 2


def _sc_body(src_hbm, dst_hbm, ns_hbm, nt_hbm, z_hbm, g_hbm, b_hbm, zeros_hbm,
             out_hbm,
             idx_s0, idx_d0, idx_s1, idx_d1, idx_s2, idx_d2, idx_s3, idx_d3,
             rows_s0, rows_t0, rows_z0, out_buf0,
             rows_s1, rows_t1, rows_z1, out_buf1,
             gb_buf, agg,
             sem_i0, sem_i1, sem_i2, sem_i3,
             sem_s0, sem_t0, sem_z0, sem_s1, sem_t1, sem_z1):
    cid = lax.axis_index("c")
    sid = lax.axis_index("s")
    wid = cid * NS + sid
    ebase = wid * EPT

    idxb = ((idx_s0, idx_d0, sem_i0), (idx_s1, idx_d1, sem_i1),
            (idx_s2, idx_d2, sem_i2), (idx_s3, idx_d3, sem_i3))
    rowb = ((rows_s0, rows_t0, rows_z0, out_buf0, sem_s0, sem_t0, sem_z0),
            (rows_s1, rows_t1, rows_z1, out_buf1, sem_s1, sem_t1, sem_z1))

    pltpu.sync_copy(zeros_hbm.at[pl.ds(sid * RPT, RPT)],
                    agg.at[pl.ds(sid * RPT, RPT)])
    pltpu.sync_copy(g_hbm, gb_buf.at[0])
    pltpu.sync_copy(b_hbm, gb_buf.at[1])
    gvec = tuple(gb_buf[0, pl.ds(k * 16, 16)] for k in range(8))
    bvec = tuple(gb_buf[1, pl.ds(k * 16, 16)] for k in range(8))

    def issue_idx(j, q):
        idx_s, idx_d, sem_i = idxb[q]
        base = ebase + j * C
        pltpu.async_copy(src_hbm.at[pl.ds(base, C)], idx_s, sem_i)
        pltpu.async_copy(dst_hbm.at[pl.ds(base, C)], idx_d, sem_i)

    def wait_idx(j, q):
        idx_s, idx_d, sem_i = idxb[q]
        base = ebase + j * C
        pltpu.make_async_copy(src_hbm.at[pl.ds(base, C)], idx_s, sem_i).wait()
        pltpu.make_async_copy(dst_hbm.at[pl.ds(base, C)], idx_d, sem_i).wait()

    def issue_gathers(j, q, b):
        idx_s, idx_d, _ = idxb[q]
        rows_s, rows_t, rows_z, _, sem_s, sem_t, sem_z = rowb[b]
        pltpu.async_copy(ns_hbm.at[idx_s], rows_s, sem_s)
        pltpu.async_copy(nt_hbm.at[idx_d], rows_t, sem_t)
        pltpu.async_copy(z_hbm.at[pl.ds(ebase + j * C, C)], rows_z, sem_z)

    def wait_gathers(j, q, b):
        idx_s, idx_d, _ = idxb[q]
        rows_s, rows_t, rows_z, _, sem_s, sem_t, sem_z = rowb[b]
        pltpu.make_async_copy(ns_hbm.at[idx_s], rows_s, sem_s).wait()
        pltpu.make_async_copy(nt_hbm.at[idx_d], rows_t, sem_t).wait()
        pltpu.make_async_copy(z_hbm.at[pl.ds(ebase + j * C, C)], rows_z,
                              sem_z).wait()

    def edge_chunk(q, b):
        _, idx_d, _ = idxb[q]
        rows_s, rows_t, rows_z, out_buf, _, _, _ = rowb[b]

        @plsc.parallel_loop(0, C, step=1, unroll=_UNROLL)
        def edge_group(e):
            xs = []
            for k in range(8):
                sl = pl.ds(k * 16, 16)
                x = rows_s[e, sl] + rows_t[e, sl] + rows_z[e, sl]
                xs.append(x / (1.0 + jnp.exp(-x)))
            tot = xs[0]
            sq = xs[0] * xs[0]
            for k in range(1, 8):
                tot = tot + xs[k]
                sq = sq + xs[k] * xs[k]
            for sh in (8, 4, 2, 1):
                perm = lax.iota(jnp.int32, 16) ^ sh
                tot = tot + _lane_gather(tot, perm)
                sq = sq + _lane_gather(sq, perm)
            mean = tot * (1.0 / D)
            ex2 = sq * (1.0 / D)
            var = ex2 - mean * mean + _LN_EPS
            bits = lax.bitcast_convert_type(var, jnp.int32)
            r = lax.bitcast_convert_type(
                jnp.int32(0x5F3759DF) - lax.shift_right_arithmetic(bits, 1),
                jnp.float32)
            for _ in range(2):
                r = r * (1.5 - 0.5 * var * r * r)
            for k in range(8):
                sl = pl.ds(k * 16, 16)
                out_buf[e, sl] = ((xs[k] - mean) * r) * gvec[k] + bvec[k]

        pltpu.sync_copy(out_buf, agg.at[idx_d], add=True)

    for q in range(4):
        issue_idx(q, q)
    wait_idx(0, 0)
    issue_gathers(0, 0, 0)
    plsc.subcore_barrier()

    LAST = NCHUNK - 1

    def quad_body(qq, carry):
        j0 = qq * 4
        for pos in range(4):
            j = j0 + pos
            b = pos % 2
            wait_idx(j + 1, (pos + 1) % 4)
            issue_gathers(j + 1, (pos + 1) % 4, 1 - b)
            wait_gathers(j, pos, b)
            edge_chunk(pos, b)
            issue_idx(jnp.minimum(j + 4, LAST), pos)
        return carry

    lax.fori_loop(0, NCHUNK // 4, quad_body, 0)

    wait_idx(LAST, 1)
    issue_gathers(LAST, 1, 1)
    wait_gathers(LAST - 1, 0, 0)
    edge_chunk(0, 0)
    wait_gathers(LAST, 1, 1)
    edge_chunk(1, 1)
    wait_idx(LAST, 2)
    wait_idx(LAST, 3)

    plsc.subcore_barrier()
    pltpu.sync_copy(agg.at[pl.ds(sid * RPT, RPT)],
                    out_hbm.at[cid, pl.ds(sid * RPT, RPT)])


_sc_call = pl.kernel(
    _sc_body,
    out_type=jax.ShapeDtypeStruct((NC, N_PAD, D), jnp.float32),
    mesh=plsc.VectorSubcoreMesh(core_axis_name="c", subcore_axis_name="s"),
    scratch_types=(
        [pltpu.VMEM((C,), jnp.int32)] * 8
        + [pltpu.VMEM((C, D), jnp.float32)] * 8
        + [pltpu.VMEM((2, D), jnp.float32),
           pltpu.VMEM_SHARED((N_PAD, D), jnp.float32)]
        + [pltpu.SemaphoreType.DMA] * 10
    ),
)


def _final_body(p_ref, ne_ref, we_ref, wt_ref, g_ref, b_ref, out_ref):
    aggv = p_ref[0, :N, :] + p_ref[1, :N, :]
    t = (jnp.dot(aggv, we_ref[...], preferred_element_type=jnp.float32)
         + jnp.dot(ne_ref[...], wt_ref[...], preferred_element_type=jnp.float32))
    t = t / (1.0 + jnp.exp(-t))
    mu = jnp.mean(t, axis=1, keepdims=True)
    d = t - mu
    var = jnp.mean(d * d, axis=1, keepdims=True)
    out_ref[...] = d * lax.rsqrt(var + _LN_EPS) * g_ref[...] + b_ref[...]


def _final(parts, node_emb, w_e2t, w_t2t, g2, b2):
    return pl.pallas_call(
        _final_body,
        out_shape=jax.ShapeDtypeStruct((N, D), jnp.float32),
    )(parts, node_emb, w_e2t, w_t2t, g2, b2)


def kernel(node_emb, edge_emb, edge_index, W_s2e, W_t2e, W_e2e, W_e2t, W_t2t,
           g1, b1, g2, b2):
    src = edge_index[0]
    dst = edge_index[1]
    z, node_s, node_t = _projections(edge_emb, W_e2e, node_emb, W_s2e, W_t2e)
    zeros = jnp.zeros((N_PAD, D), jnp.float32)
    parts = _sc_call(src, dst, node_s, node_t, z, g1, b1, zeros)
    return _final(parts, node_emb, W_e2t, W_t2t,
                  g2.reshape(1, D), b2.reshape(1, D))


# speedup vs baseline: 1.0990x; 1.0843x over previous
import functools

import jax
import jax.numpy as jnp
from jax import lax
from jax.experimental import pallas as pl
from jax.experimental.pallas import tpu as pltpu
from jax.experimental.pallas import tpu_sc as plsc

N = 10000
E = 320000
D = 128

NC = 2
NS = 16
NW = NC * NS
EPT = E // NW
C = 40
NCHUNK = EPT // C
RPT = 632
N_PAD = RPT * NS

_LN_EPS = 1e-5


_BE = 3200


def _proj_body(ee_ref, we_ref, ne_ref, ws_ref, wt_ref, z_ref, ns_ref, nt_ref):
    z_ref[...] = jnp.dot(ee_ref[...], we_ref[...],
                         preferred_element_type=jnp.float32)

    @pl.when(pl.program_id(0) == 0)
    def _():
        x = ne_ref[...]
        ns_ref[...] = jnp.dot(x, ws_ref[...], preferred_element_type=jnp.float32)
        nt_ref[...] = jnp.dot(x, wt_ref[...], preferred_element_type=jnp.float32)


def _projections(edge_emb, w_e, node_emb, w_s, w_t):
    full = lambda i: (0, 0)
    return pl.pallas_call(
        _proj_body,
        grid=(E // _BE,),
        in_specs=[
            pl.BlockSpec((_BE, D), lambda i: (i, 0)),
            pl.BlockSpec((D, D), full),
            pl.BlockSpec((N, D), full),
            pl.BlockSpec((D, D), full),
            pl.BlockSpec((D, D), full),
        ],
        out_specs=[
            pl.BlockSpec((_BE, D), lambda i: (i, 0)),
            pl.BlockSpec((N, D), full),
            pl.BlockSpec((N, D), full),
        ],
        out_shape=[
            jax.ShapeDtypeStruct((E, D), jnp.float32),
            jax.ShapeDtypeStruct((N, D), jnp.float32),
            jax.ShapeDtypeStruct((N, D), jnp.float32),
        ],
    )(edge_emb, w_e, node_emb, w_s, w_t)


def _lane_gather(v, perm):
    dnums = lax.GatherDimensionNumbers(
        offset_dims=(), collapsed_slice_dims=(0,), start_index_map=(0,))
    return lax.gather(v, perm[:, None], dnums, (1,),
                      mode=lax.GatherScatterMode.PROMISE_IN_BOUNDS)


_UNROLL = 2


def _sc_body(src_hbm, dst_hbm, ns_hbm, nt_hbm, z_hbm, g_hbm, b_hbm, zeros_hbm,
             out_hbm,
             idx_s0, idx_d0, idx_s1, idx_d1, idx_s2, idx_d2, idx_s3, idx_d3,
             rows_s0, rows_t0, rows_z0, out_buf0,
             rows_s1, rows_t1, rows_z1, out_buf1,
             gb_buf, agg,
             sem_i0, sem_i1, sem_i2, sem_i3,
             sem_s0, sem_t0, sem_z0, sem_s1, sem_t1, sem_z1,
             sem_o0, sem_o1):
    cid = lax.axis_index("c")
    sid = lax.axis_index("s")
    wid = cid * NS + sid
    ebase = wid * EPT

    idxb = ((idx_s0, idx_d0, sem_i0), (idx_s1, idx_d1, sem_i1),
            (idx_s2, idx_d2, sem_i2), (idx_s3, idx_d3, sem_i3))
    rowb = ((rows_s0, rows_t0, rows_z0, out_buf0, sem_s0, sem_t0, sem_z0),
            (rows_s1, rows_t1, rows_z1, out_buf1, sem_s1, sem_t1, sem_z1))
    semo = (sem_o0, sem_o1)

    pltpu.sync_copy(zeros_hbm.at[pl.ds(sid * RPT, RPT)],
                    agg.at[pl.ds(sid * RPT, RPT)])
    pltpu.sync_copy(g_hbm, gb_buf.at[0])
    pltpu.sync_copy(b_hbm, gb_buf.at[1])
    gvec = tuple(gb_buf[0, pl.ds(k * 16, 16)] for k in range(8))
    bvec = tuple(gb_buf[1, pl.ds(k * 16, 16)] for k in range(8))

    def issue_idx(j, q):
        idx_s, idx_d, sem_i = idxb[q]
        base = ebase + j * C
        pltpu.async_copy(src_hbm.at[pl.ds(base, C)], idx_s, sem_i)
        pltpu.async_copy(dst_hbm.at[pl.ds(base, C)], idx_d, sem_i)

    def wait_idx(j, q):
        idx_s, idx_d, sem_i = idxb[q]
        base = ebase + j * C
        pltpu.make_async_copy(src_hbm.at[pl.ds(base, C)], idx_s, sem_i).wait()
        pltpu.make_async_copy(dst_hbm.at[pl.ds(base, C)], idx_d, sem_i).wait()

    def issue_gathers(j, q, b):
        idx_s, idx_d, _ = idxb[q]
        rows_s, rows_t, rows_z, _, sem_s, sem_t, sem_z = rowb[b]
        pltpu.async_copy(ns_hbm.at[idx_s], rows_s, sem_s)
        pltpu.async_copy(nt_hbm.at[idx_d], rows_t, sem_t)
        pltpu.async_copy(z_hbm.at[pl.ds(ebase + j * C, C)], rows_z, sem_z)

    def wait_gathers(j, q, b):
        idx_s, idx_d, _ = idxb[q]
        rows_s, rows_t, rows_z, _, sem_s, sem_t, sem_z = rowb[b]
        pltpu.make_async_copy(ns_hbm.at[idx_s], rows_s, sem_s).wait()
        pltpu.make_async_copy(nt_hbm.at[idx_d], rows_t, sem_t).wait()
        pltpu.make_async_copy(z_hbm.at[pl.ds(ebase + j * C, C)], rows_z,
                              sem_z).wait()

    def edge_chunk(q, b):
        _, idx_d, _ = idxb[q]
        rows_s, rows_t, rows_z, out_buf, _, _, _ = rowb[b]

        @plsc.parallel_loop(0, C, step=1, unroll=_UNROLL)
        def edge_group(e):
            xs = []
            for k in range(8):
                sl = pl.ds(k * 16, 16)
                x = rows_s[e, sl] + rows_t[e, sl] + rows_z[e, sl]
                xs.append(x / (1.0 + jnp.exp(-x)))
            tot = xs[0]
            sq = xs[0] * xs[0]
            for k in range(1, 8):
                tot = tot + xs[k]
                sq = sq + xs[k] * xs[k]
            for sh in (8, 4, 2, 1):
                perm = lax.iota(jnp.int32, 16) ^ sh
                tot = tot + _lane_gather(tot, perm)
                sq = sq + _lane_gather(sq, perm)
            mean = tot * (1.0 / D)
            ex2 = sq * (1.0 / D)
            var = ex2 - mean * mean + _LN_EPS
            bits = lax.bitcast_convert_type(var, jnp.int32)
            r = lax.bitcast_convert_type(
                jnp.int32(0x5F3759DF) - lax.shift_right_arithmetic(bits, 1),
                jnp.float32)
            for _ in range(2):
                r = r * (1.5 - 0.5 * var * r * r)
            for k in range(8):
                sl = pl.ds(k * 16, 16)
                out_buf[e, sl] = ((xs[k] - mean) * r) * gvec[k] + bvec[k]

        pltpu.async_copy(out_buf, agg.at[idx_d], semo[b], add=True)

    def wait_scatter(q, b):
        _, idx_d, _ = idxb[q]
        _, _, _, out_buf, _, _, _ = rowb[b]
        pltpu.make_async_copy(out_buf, agg.at[idx_d], semo[b]).wait()

    issue_idx(0, 0)
    issue_idx(1, 1)
    wait_idx(0, 0)
    issue_gathers(0, 0, 0)
    plsc.subcore_barrier()

    LAST = NCHUNK - 1

    def quad_body(qq, carry):
        j0 = qq * 4
        for pos in range(4):
            j = j0 + pos
            b = pos % 2
            wait_idx(j + 1, (pos + 1) % 4)
            issue_gathers(j + 1, (pos + 1) % 4, 1 - b)
            wait_gathers(j, pos, b)
            if pos < 2:
                @pl.when(qq > 0)
                def _():
                    wait_scatter((pos + 2) % 4, b)
            else:
                wait_scatter((pos + 2) % 4, b)
            issue_idx(j + 2, (pos + 2) % 4)
            edge_chunk(pos, b)
        return carry

    lax.fori_loop(0, NCHUNK // 4, quad_body, 0)

    wait_idx(LAST, 1)
    issue_gathers(LAST, 1, 1)
    wait_gathers(LAST - 1, 0, 0)
    wait_scatter(0, 0)
    edge_chunk(0, 0)
    wait_gathers(LAST, 1, 1)
    wait_scatter(1, 1)
    edge_chunk(1, 1)
    wait_scatter(0, 0)
    wait_scatter(1, 1)

    plsc.subcore_barrier()
    pltpu.sync_copy(agg.at[pl.ds(sid * RPT, RPT)],
                    out_hbm.at[cid, pl.ds(sid * RPT, RPT)])


_sc_call = pl.kernel(
    _sc_body,
    out_type=jax.ShapeDtypeStruct((NC, N_PAD, D), jnp.float32),
    mesh=plsc.VectorSubcoreMesh(core_axis_name="c", subcore_axis_name="s"),
    scratch_types=(
        [pltpu.VMEM((C,), jnp.int32)] * 8
        + [pltpu.VMEM((C, D), jnp.float32)] * 8
        + [pltpu.VMEM((2, D), jnp.float32),
           pltpu.VMEM_SHARED((N_PAD, D), jnp.float32)]
        + [pltpu.SemaphoreType.DMA] * 12
    ),
)


def _final_body(p_ref, ne_ref, we_ref, wt_ref, g_ref, b_ref, out_ref):
    aggv = p_ref[0, :N, :] + p_ref[1, :N, :]
    t = (jnp.dot(aggv, we_ref[...], preferred_element_type=jnp.float32)
         + jnp.dot(ne_ref[...], wt_ref[...], preferred_element_type=jnp.float32))
    t = t / (1.0 + jnp.exp(-t))
    mu = jnp.mean(t, axis=1, keepdims=True)
    d = t - mu
    var = jnp.mean(d * d, axis=1, keepdims=True)
    out_ref[...] = d * lax.rsqrt(var + _LN_EPS) * g_ref[...] + b_ref[...]


def _final(parts, node_emb, w_e2t, w_t2t, g2, b2):
    return pl.pallas_call(
        _final_body,
        out_shape=jax.ShapeDtypeStruct((N, D), jnp.float32),
    )(parts, node_emb, w_e2t, w_t2t, g2, b2)


def kernel(node_emb, edge_emb, edge_index, W_s2e, W_t2e, W_e2e, W_e2t, W_t2t,
           g1, b1, g2, b2):
    src = edge_index[0]
    dst = edge_index[1]
    z, node_s, node_t = _projections(edge_emb, W_e2e, node_emb, W_s2e, W_t2e)
    zeros = jnp.zeros((N_PAD, D), jnp.float32)
    parts = _sc_call(src, dst, node_s, node_t, z, g1, b1, zeros)
    return _final(parts, node_emb, W_e2t, W_t2t,
                  g2.reshape(1, D), b2.reshape(1, D))
